# Initial kernel scaffold; baseline (speedup 1.0000x reference)
import jax
import jax.numpy as jnp
from jax.experimental import pallas as pl


def kernel(edge_index, W1, al1, ar1, b1, W2, al2, ar2, b2, W3, al3, ar3, b3, Wc, bc):
    raise NotImplementedError("write your pallas kernel here")


# speedup vs baseline: 21.9298x; 21.9298x over previous
import functools

import jax
import jax.numpy as jnp
from jax import lax
from jax.experimental import pallas as pl
from jax.experimental.pallas import tpu as pltpu
from jax.experimental.pallas import tpu_sc as plsc

N_NODES = 100000
N_EDGES = 3200000
HID = 16
N_CLASSES = 10

NCORE = 2
NSUB = 16
NW = NCORE * NSUB
EPW = N_EDGES // NW
CHUNK = 800
SUB = 80
NSLICE = CHUNK // SUB
NCHUNK = EPW // CHUNK

RPT = 6256
NPAD = RPT * NSUB
ZROWS = 368
NSTRIP = RPT // ZROWS

RBLK = 2176
GRID = NPAD // RBLK

NEG = -3.0e38


def _lrelu(x):
    return jnp.where(x >= 0.0, x, 0.2 * x)


def _wid():
    return lax.axis_index("s") * NCORE + lax.axis_index("c")


_MESH = plsc.VectorSubcoreMesh(core_axis_name="c", subcore_axis_name="s")


@functools.partial(
    pl.kernel,
    mesh=_MESH,
    compiler_params=pltpu.CompilerParams(needs_layout_passes=False, use_tc_tiling_on_sc=False),
    out_type=jax.ShapeDtypeStruct((NCORE * NPAD,), jnp.float32),
    scratch_types=[
        pltpu.VMEM((NSLICE, SUB), jnp.int32),
        pltpu.VMEM((SUB,), jnp.float32),
        pltpu.VMEM((RPT,), jnp.float32),
        pltpu.VMEM_SHARED((NPAD,), jnp.float32),
    ],
)
def _deg_kernel(dst_hbm, out_hbm, dsti, ones_v, zb, deg_sp):
    c = lax.axis_index("c")
    s = lax.axis_index("s")
    roff = s * RPT

    def _fill(i, _):
        zb[pl.ds(i * 16, 16)] = jnp.zeros((16,), jnp.float32)
        return 0

    lax.fori_loop(0, RPT // 16, _fill, 0)
    for i in range(SUB // 16):
        ones_v[pl.ds(i * 16, 16)] = jnp.ones((16,), jnp.float32)
    pltpu.sync_copy(zb, deg_sp.at[pl.ds(roff, RPT)])
    plsc.subcore_barrier()

    ebase = _wid() * EPW

    def _chunk(k, _):
        base = ebase + k * CHUNK
        for t in range(NSLICE):
            pltpu.sync_copy(dst_hbm.at[pl.ds(base + t * SUB, SUB)], dsti.at[t])
            pltpu.sync_copy(ones_v, deg_sp.at[dsti.at[t]], add=True)
        return 0

    lax.fori_loop(0, NCHUNK, _chunk, 0)
    plsc.subcore_barrier()
    pltpu.sync_copy(deg_sp.at[pl.ds(roff, RPT)], zb)
    pltpu.sync_copy(zb, out_hbm.at[pl.ds(c * NPAD + roff, RPT)])


@functools.partial(
    pl.kernel,
    mesh=_MESH,
    compiler_params=pltpu.CompilerParams(needs_layout_passes=False, use_tc_tiling_on_sc=False),
    out_type=(
        jax.ShapeDtypeStruct((NCORE * NPAD,), jnp.float32),
        jax.ShapeDtypeStruct((N_EDGES,), jnp.float32),
    ),
    scratch_types=[
        pltpu.VMEM((NSLICE, SUB), jnp.int32),
        pltpu.VMEM((CHUNK,), jnp.int32),
        pltpu.VMEM((CHUNK,), jnp.float32),
        pltpu.VMEM((NPAD,), jnp.float32),
        pltpu.VMEM((RPT,), jnp.float32),
        pltpu.VMEM((RPT,), jnp.float32),
        pltpu.HBM((NW * NPAD,), jnp.float32),
    ],
)
def _maxel_kernel(src_hbm, dst_hbm, el_hbm, pm_hbm, elsrc_hbm,
                  srci, dstv, elv, mx, acc, tmp, stage):
    c = lax.axis_index("c")
    s = lax.axis_index("s")

    def _fill(i, _):
        mx[pl.ds(i * 16, 16)] = jnp.full((16,), NEG, jnp.float32)
        return 0

    lax.fori_loop(0, NPAD // 16, _fill, 0)

    ebase = _wid() * EPW

    def _chunk(k, _):
        base = ebase + k * CHUNK
        pltpu.sync_copy(dst_hbm.at[pl.ds(base, CHUNK)], dstv)
        for t in range(NSLICE):
            pltpu.sync_copy(src_hbm.at[pl.ds(base + t * SUB, SUB)], srci.at[t])
            pltpu.sync_copy(el_hbm.at[srci.at[t]],
                            elv.at[pl.ds(t * SUB, SUB)])
        pltpu.sync_copy(elv, elsrc_hbm.at[pl.ds(base, CHUNK)])

        def _vec(j, _):
            idx = dstv[pl.ds(j * 16, 16)]
            ev = elv[pl.ds(j * 16, 16)]
            cur = plsc.load_gather(mx, [idx])

            def _cond(carry):
                return jnp.any(ev > carry)

            def _body(carry):
                plsc.store_scatter(mx, [idx], ev, mask=ev > carry)
                return plsc.load_gather(mx, [idx])

            lax.while_loop(_cond, _body, cur)
            return 0

        lax.fori_loop(0, CHUNK // 16, _vec, 0)
        return 0

    lax.fori_loop(0, NCHUNK, _chunk, 0)

    pltpu.sync_copy(mx, stage.at[pl.ds((c * NSUB + s) * NPAD, NPAD)])
    plsc.subcore_barrier()

    roff = s * RPT
    for t in range(NSUB):
        if t == 0:
            pltpu.sync_copy(stage.at[pl.ds(c * NSUB * NPAD + roff, RPT)], acc)
        else:
            pltpu.sync_copy(stage.at[pl.ds((c * NSUB + t) * NPAD + roff, RPT)],
                            tmp)

            def _red(j, _):
                acc[pl.ds(j * 16, 16)] = jnp.maximum(
                    acc[pl.ds(j * 16, 16)], tmp[pl.ds(j * 16, 16)])
                return 0

            lax.fori_loop(0, RPT // 16, _red, 0)
    pltpu.sync_copy(acc, pm_hbm.at[pl.ds(c * NPAD + roff, RPT)])


@functools.partial(
    pl.kernel,
    mesh=_MESH,
    compiler_params=pltpu.CompilerParams(needs_layout_passes=False, use_tc_tiling_on_sc=False),
    out_type=(
        jax.ShapeDtypeStruct((NCORE * NPAD, HID), jnp.float32),
        jax.ShapeDtypeStruct((NCORE * NPAD,), jnp.float32),
    ),
    scratch_types=[
        pltpu.VMEM((NSLICE, SUB), jnp.int32),
        pltpu.VMEM((NSLICE, SUB), jnp.int32),
        pltpu.VMEM((CHUNK,), jnp.float32),
        pltpu.VMEM((CHUNK,), jnp.float32),
        pltpu.VMEM((CHUNK,), jnp.float32),
        pltpu.VMEM((CHUNK,), jnp.float32),
        pltpu.VMEM((CHUNK, HID), jnp.float32),
        pltpu.VMEM_SHARED((NPAD, HID), jnp.float32),
        pltpu.VMEM_SHARED((NPAD,), jnp.float32),
    ],
)
def _edge_kernel(src_hbm, dst_hbm, elsrc_hbm, er_hbm, emax_hbm, feat_hbm,
                 num_hbm, den_hbm,
                 srci, dsti, elsv, erg, emg, wv, featg,
                 num_sp, den_sp):
    c = lax.axis_index("c")
    s = lax.axis_index("s")
    roff = s * RPT

    def _fillzr(i, _):
        featg[i] = jnp.zeros((HID,), jnp.float32)
        return 0

    lax.fori_loop(0, CHUNK, _fillzr, 0)

    def _fillz(i, _):
        wv[pl.ds(i * 16, 16)] = jnp.zeros((16,), jnp.float32)
        return 0

    lax.fori_loop(0, CHUNK // 16, _fillz, 0)

    for t in range(7):
        pltpu.sync_copy(wv, den_sp.at[pl.ds(roff + t * CHUNK, CHUNK)])
        pltpu.sync_copy(featg, num_sp.at[pl.ds(roff + t * CHUNK, CHUNK)])
    pltpu.sync_copy(wv.at[pl.ds(0, 656)],
                    den_sp.at[pl.ds(roff + 7 * CHUNK, 656)])
    pltpu.sync_copy(featg.at[pl.ds(0, 656)],
                    num_sp.at[pl.ds(roff + 7 * CHUNK, 656)])
    plsc.subcore_barrier()

    ebase = _wid() * EPW

    def _chunk(k, _):
        base = ebase + k * CHUNK
        pltpu.sync_copy(elsrc_hbm.at[pl.ds(base, CHUNK)], elsv)
        for t in range(NSLICE):
            pltpu.sync_copy(src_hbm.at[pl.ds(base + t * SUB, SUB)], srci.at[t])
            pltpu.sync_copy(dst_hbm.at[pl.ds(base + t * SUB, SUB)], dsti.at[t])
            pltpu.sync_copy(er_hbm.at[dsti.at[t]], erg.at[pl.ds(t * SUB, SUB)])
            pltpu.sync_copy(emax_hbm.at[dsti.at[t]],
                            emg.at[pl.ds(t * SUB, SUB)])
            pltpu.sync_copy(feat_hbm.at[srci.at[t]],
                            featg.at[pl.ds(t * SUB, SUB)])

        def _w(j, _):
            sl = pl.ds(j * 16, 16)
            t0 = elsv[sl] + erg[sl]
            e = jnp.where(t0 >= 0.0, t0, 0.2 * t0)
            w = jnp.exp(e - emg[sl])
            wv[sl] = w
            for k in range(16):
                kk = j * 16 + k
                featg[kk] = featg[kk] * w[k]
            return 0

        lax.fori_loop(0, CHUNK // 16, _w, 0)

        for t in range(NSLICE):
            pltpu.sync_copy(featg.at[pl.ds(t * SUB, SUB)],
                            num_sp.at[dsti.at[t]], add=True)
            pltpu.sync_copy(wv.at[pl.ds(t * SUB, SUB)],
                            den_sp.at[dsti.at[t]], add=True)
        return 0

    lax.fori_loop(0, NCHUNK, _chunk, 0)
    plsc.subcore_barrier()

    for t in range(7):
        pltpu.sync_copy(den_sp.at[pl.ds(roff + t * CHUNK, CHUNK)], wv)
        pltpu.sync_copy(wv, den_hbm.at[pl.ds(c * NPAD + roff + t * CHUNK,
                                             CHUNK)])
        pltpu.sync_copy(num_sp.at[pl.ds(roff + t * CHUNK, CHUNK)], featg)
        pltpu.sync_copy(featg, num_hbm.at[pl.ds(c * NPAD + roff + t * CHUNK,
                                                CHUNK)])
    pltpu.sync_copy(den_sp.at[pl.ds(roff + 7 * CHUNK, 656)],
                    wv.at[pl.ds(0, 656)])
    pltpu.sync_copy(wv.at[pl.ds(0, 656)],
                    den_hbm.at[pl.ds(c * NPAD + roff + 7 * CHUNK, 656)])
    pltpu.sync_copy(num_sp.at[pl.ds(roff + 7 * CHUNK, 656)],
                    featg.at[pl.ds(0, 656)])
    pltpu.sync_copy(featg.at[pl.ds(0, 656)],
                    num_hbm.at[pl.ds(c * NPAD + roff + 7 * CHUNK, 656)])


def _dense1_body(degp, w1, al, ar, feat_o, el_o, er_o):
    deg = degp[0] + degp[1]
    f = deg * w1[...]
    feat_o[...] = f
    el_o[...] = jnp.sum(f * al[...], axis=1, keepdims=True)
    er_o[...] = jnp.sum(f * ar[...], axis=1, keepdims=True)


def _dense1(degp, w1, al, ar):
    return pl.pallas_call(
        _dense1_body,
        grid=(GRID,),
        in_specs=[
            pl.BlockSpec((2, RBLK, 1), lambda i: (0, i, 0)),
            pl.BlockSpec((1, HID), lambda i: (0, 0)),
            pl.BlockSpec((1, HID), lambda i: (0, 0)),
            pl.BlockSpec((1, HID), lambda i: (0, 0)),
        ],
        out_specs=[
            pl.BlockSpec((RBLK, HID), lambda i: (i, 0)),
            pl.BlockSpec((RBLK, 1), lambda i: (i, 0)),
            pl.BlockSpec((RBLK, 1), lambda i: (i, 0)),
        ],
        out_shape=[
            jax.ShapeDtypeStruct((NPAD, HID), jnp.float32),
            jax.ShapeDtypeStruct((NPAD, 1), jnp.float32),
            jax.ShapeDtypeStruct((NPAD, 1), jnp.float32),
        ],
    )(degp, w1, al, ar)


def _dense23_body(nump, denp, b, w, al, ar, feat_o, el_o, er_o):
    num = nump[0] + nump[1]
    den = denp[0] + denp[1]
    safe = jnp.where(den > 0.0, den, 1.0)
    rst = jnp.where(den > 0.0, num / safe, 0.0) + b[...]
    h = jnp.maximum(rst, 0.0)
    f = jnp.dot(h, w[...], preferred_element_type=jnp.float32)
    feat_o[...] = f
    el_o[...] = jnp.sum(f * al[...], axis=1, keepdims=True)
    er_o[...] = jnp.sum(f * ar[...], axis=1, keepdims=True)


def _dense23(nump, denp, b, w, al, ar):
    return pl.pallas_call(
        _dense23_body,
        grid=(GRID,),
        in_specs=[
            pl.BlockSpec((2, RBLK, HID), lambda i: (0, i, 0)),
            pl.BlockSpec((2, RBLK, 1), lambda i: (0, i, 0)),
            pl.BlockSpec((1, HID), lambda i: (0, 0)),
            pl.BlockSpec((HID, HID), lambda i: (0, 0)),
            pl.BlockSpec((1, HID), lambda i: (0, 0)),
            pl.BlockSpec((1, HID), lambda i: (0, 0)),
        ],
        out_specs=[
            pl.BlockSpec((RBLK, HID), lambda i: (i, 0)),
            pl.BlockSpec((RBLK, 1), lambda i: (i, 0)),
            pl.BlockSpec((RBLK, 1), lambda i: (i, 0)),
        ],
        out_shape=[
            jax.ShapeDtypeStruct((NPAD, HID), jnp.float32),
            jax.ShapeDtypeStruct((NPAD, 1), jnp.float32),
            jax.ShapeDtypeStruct((NPAD, 1), jnp.float32),
        ],
    )(nump, denp, b, w, al, ar)


def _emax_body(pm, er, out):
    m = jnp.maximum(pm[0], pm[1]) + er[...]
    out[...] = jnp.where(m >= 0.0, m, 0.2 * m)


def _emax(pm, er):
    return pl.pallas_call(
        _emax_body,
        grid=(GRID,),
        in_specs=[
            pl.BlockSpec((2, RBLK, 1), lambda i: (0, i, 0)),
            pl.BlockSpec((RBLK, 1), lambda i: (i, 0)),
        ],
        out_specs=pl.BlockSpec((RBLK, 1), lambda i: (i, 0)),
        out_shape=jax.ShapeDtypeStruct((NPAD, 1), jnp.float32),
    )(pm, er)


def _final_body(nump, denp, b, wc, bc, out, acc):
    i = pl.program_id(0)
    num = nump[0] + nump[1]
    den = denp[0] + denp[1]
    safe = jnp.where(den > 0.0, den, 1.0)
    h = jnp.maximum(jnp.where(den > 0.0, num / safe, 0.0) + b[...], 0.0)
    rows = jax.lax.broadcasted_iota(jnp.int32, (RBLK, 1), 0) + i * RBLK
    h = jnp.where(rows < N_NODES, h, 0.0)

    @pl.when(i == 0)
    def _():
        acc[...] = jnp.zeros_like(acc)

    acc[...] += jnp.sum(h, axis=0, keepdims=True)

    @pl.when(i == GRID - 1)
    def _():
        hg = acc[...] * (1.0 / N_NODES)
        out[...] = (jnp.dot(hg, wc[...], preferred_element_type=jnp.float32)
                    + bc[...])


def _final(nump, denp, b, wc, bc):
    return pl.pallas_call(
        _final_body,
        grid=(GRID,),
        in_specs=[
            pl.BlockSpec((2, RBLK, HID), lambda i: (0, i, 0)),
            pl.BlockSpec((2, RBLK, 1), lambda i: (0, i, 0)),
            pl.BlockSpec((1, HID), lambda i: (0, 0)),
            pl.BlockSpec((HID, N_CLASSES), lambda i: (0, 0)),
            pl.BlockSpec((1, N_CLASSES), lambda i: (0, 0)),
        ],
        out_specs=pl.BlockSpec((1, N_CLASSES), lambda i: (0, 0)),
        out_shape=jax.ShapeDtypeStruct((1, N_CLASSES), jnp.float32),
        scratch_shapes=[pltpu.VMEM((1, HID), jnp.float32)],
    )(nump, denp, b, wc, bc)


def _gat_edge_phase(src, dst, feat, el, er):
    pm, elsrc = _maxel_kernel(src, dst, el.reshape(NPAD))
    emax = _emax(pm.reshape(2, NPAD, 1), er)
    nump, denp = _edge_kernel(src, dst, elsrc, er.reshape(NPAD),
                              emax.reshape(NPAD), feat)
    return nump.reshape(2, NPAD, HID), denp.reshape(2, NPAD, 1)


def kernel(edge_index, W1, al1, ar1, b1, W2, al2, ar2, b2,
           W3, al3, ar3, b3, Wc, bc):
    src = edge_index[0]
    dst = edge_index[1]

    degp = _deg_kernel(dst).reshape(2, NPAD, 1)
    feat1, el1, er1 = _dense1(degp, W1, al1.reshape(1, HID),
                              ar1.reshape(1, HID))
    nump, denp = _gat_edge_phase(src, dst, feat1, el1, er1)

    feat2, el2, er2 = _dense23(nump, denp, b1.reshape(1, HID), W2,
                               al2.reshape(1, HID), ar2.reshape(1, HID))
    nump, denp = _gat_edge_phase(src, dst, feat2, el2, er2)

    feat3, el3, er3 = _dense23(nump, denp, b2.reshape(1, HID), W3,
                               al3.reshape(1, HID), ar3.reshape(1, HID))
    nump, denp = _gat_edge_phase(src, dst, feat3, el3, er3)

    return _final(nump, denp, b3.reshape(1, HID), Wc,
                  bc.reshape(1, N_CLASSES))


# speedup vs baseline: 59.9053x; 2.7317x over previous
import functools

import jax
import jax.numpy as jnp
from jax import lax
from jax.experimental import pallas as pl
from jax.experimental.pallas import tpu as pltpu
from jax.experimental.pallas import tpu_sc as plsc

N_NODES = 100000
N_EDGES = 3200000
HID = 16
N_CLASSES = 10

NCORE = 2
NSUB = 16
NW = NCORE * NSUB
EPW = N_EDGES // NW
CHUNK = 800
SUB = 80
NSLICE = CHUNK // SUB
NCHUNK = EPW // CHUNK

RPT = 6256
NPAD = RPT * NSUB
ZROWS = 368
NSTRIP = RPT // ZROWS

RBLK = 2176
GRID = NPAD // RBLK

NEG = -3.0e38


def _lrelu(x):
    return jnp.where(x >= 0.0, x, 0.2 * x)


def _wid():
    return lax.axis_index("s") * NCORE + lax.axis_index("c")


_MESH = plsc.VectorSubcoreMesh(core_axis_name="c", subcore_axis_name="s")


@functools.partial(
    pl.kernel,
    mesh=_MESH,
    compiler_params=pltpu.CompilerParams(needs_layout_passes=False, use_tc_tiling_on_sc=False),
    out_type=jax.ShapeDtypeStruct((NCORE * NPAD,), jnp.float32),
    scratch_types=[
        pltpu.VMEM((1, CHUNK), jnp.int32),
        pltpu.VMEM((CHUNK,), jnp.float32),
        pltpu.VMEM((RPT,), jnp.float32),
        pltpu.VMEM_SHARED((NPAD,), jnp.float32),
    ],
)
def _deg_kernel(dst_hbm, out_hbm, dsti, ones_v, zb, deg_sp):
    c = lax.axis_index("c")
    s = lax.axis_index("s")
    roff = s * RPT

    def _fill(i, _):
        zb[pl.ds(i * 16, 16)] = jnp.zeros((16,), jnp.float32)
        return 0

    lax.fori_loop(0, RPT // 16, _fill, 0)

    def _fill1(i, _):
        ones_v[pl.ds(i * 16, 16)] = jnp.ones((16,), jnp.float32)
        return 0

    lax.fori_loop(0, CHUNK // 16, _fill1, 0)
    pltpu.sync_copy(zb, deg_sp.at[pl.ds(roff, RPT)])
    plsc.subcore_barrier()

    ebase = _wid() * EPW

    def _chunk(k, _):
        base = ebase + k * CHUNK
        pltpu.sync_copy(dst_hbm.at[pl.ds(base, CHUNK)], dsti.at[0])
        pltpu.sync_copy(ones_v, deg_sp.at[dsti.at[0]], add=True)
        return 0

    lax.fori_loop(0, NCHUNK, _chunk, 0)
    plsc.subcore_barrier()
    pltpu.sync_copy(deg_sp.at[pl.ds(roff, RPT)], zb)
    pltpu.sync_copy(zb, out_hbm.at[pl.ds(c * NPAD + roff, RPT)])


@functools.partial(
    pl.kernel,
    mesh=_MESH,
    compiler_params=pltpu.CompilerParams(needs_layout_passes=False, use_tc_tiling_on_sc=False),
    out_type=(
        jax.ShapeDtypeStruct((NCORE * NPAD,), jnp.float32),
        jax.ShapeDtypeStruct((N_EDGES,), jnp.float32),
    ),
    scratch_types=[
        pltpu.VMEM((1, CHUNK), jnp.int32),
        pltpu.VMEM((CHUNK,), jnp.int32),
        pltpu.VMEM((CHUNK,), jnp.float32),
        pltpu.VMEM((NPAD,), jnp.float32),
        pltpu.VMEM((RPT,), jnp.float32),
        pltpu.VMEM((RPT,), jnp.float32),
        pltpu.HBM((NW * NPAD,), jnp.float32),
    ],
)
def _maxel_kernel(src_hbm, dst_hbm, el_hbm, pm_hbm, elsrc_hbm,
                  srci, dstv, elv, mx, acc, tmp, stage):
    c = lax.axis_index("c")
    s = lax.axis_index("s")

    def _fill(i, _):
        mx[pl.ds(i * 16, 16)] = jnp.full((16,), NEG, jnp.float32)
        return 0

    lax.fori_loop(0, NPAD // 16, _fill, 0)

    ebase = _wid() * EPW

    def _chunk(k, _):
        base = ebase + k * CHUNK
        pltpu.sync_copy(dst_hbm.at[pl.ds(base, CHUNK)], dstv)
        pltpu.sync_copy(src_hbm.at[pl.ds(base, CHUNK)], srci.at[0])
        pltpu.sync_copy(el_hbm.at[srci.at[0]], elv)
        pltpu.sync_copy(elv, elsrc_hbm.at[pl.ds(base, CHUNK)])

        def _vec(j, _):
            idx = dstv[pl.ds(j * 16, 16)]
            ev = elv[pl.ds(j * 16, 16)]
            cur = plsc.load_gather(mx, [idx])

            def _cond(carry):
                return jnp.any(ev > carry)

            def _body(carry):
                plsc.store_scatter(mx, [idx], ev, mask=ev > carry)
                return plsc.load_gather(mx, [idx])

            lax.while_loop(_cond, _body, cur)
            return 0

        lax.fori_loop(0, CHUNK // 16, _vec, 0)
        return 0

    lax.fori_loop(0, NCHUNK, _chunk, 0)

    pltpu.sync_copy(mx, stage.at[pl.ds((c * NSUB + s) * NPAD, NPAD)])
    plsc.subcore_barrier()

    roff = s * RPT
    for t in range(NSUB):
        if t == 0:
            pltpu.sync_copy(stage.at[pl.ds(c * NSUB * NPAD + roff, RPT)], acc)
        else:
            pltpu.sync_copy(stage.at[pl.ds((c * NSUB + t) * NPAD + roff, RPT)],
                            tmp)

            def _red(j, _):
                acc[pl.ds(j * 16, 16)] = jnp.maximum(
                    acc[pl.ds(j * 16, 16)], tmp[pl.ds(j * 16, 16)])
                return 0

            lax.fori_loop(0, RPT // 16, _red, 0)
    pltpu.sync_copy(acc, pm_hbm.at[pl.ds(c * NPAD + roff, RPT)])


@functools.partial(
    pl.kernel,
    mesh=_MESH,
    compiler_params=pltpu.CompilerParams(needs_layout_passes=False, use_tc_tiling_on_sc=False),
    out_type=(
        jax.ShapeDtypeStruct((NCORE * NPAD, HID), jnp.float32),
        jax.ShapeDtypeStruct((NCORE * NPAD,), jnp.float32),
    ),
    scratch_types=[
        pltpu.VMEM((1, CHUNK), jnp.int32),
        pltpu.VMEM((1, CHUNK), jnp.int32),
        pltpu.VMEM((CHUNK,), jnp.float32),
        pltpu.VMEM((CHUNK,), jnp.float32),
        pltpu.VMEM((CHUNK,), jnp.float32),
        pltpu.VMEM((CHUNK,), jnp.float32),
        pltpu.VMEM((CHUNK, HID), jnp.float32),
        pltpu.VMEM_SHARED((NPAD, HID), jnp.float32),
        pltpu.VMEM_SHARED((NPAD,), jnp.float32),
    ],
)
def _edge_kernel(src_hbm, dst_hbm, elsrc_hbm, er_hbm, emax_hbm, feat_hbm,
                 num_hbm, den_hbm,
                 srci, dsti, elsv, erg, emg, wv, featg,
                 num_sp, den_sp):
    c = lax.axis_index("c")
    s = lax.axis_index("s")
    roff = s * RPT

    def _fillzr(i, _):
        featg[i] = jnp.zeros((HID,), jnp.float32)
        return 0

    lax.fori_loop(0, CHUNK, _fillzr, 0)

    def _fillz(i, _):
        wv[pl.ds(i * 16, 16)] = jnp.zeros((16,), jnp.float32)
        return 0

    lax.fori_loop(0, CHUNK // 16, _fillz, 0)

    for t in range(7):
        pltpu.sync_copy(wv, den_sp.at[pl.ds(roff + t * CHUNK, CHUNK)])
        pltpu.sync_copy(featg, num_sp.at[pl.ds(roff + t * CHUNK, CHUNK)])
    pltpu.sync_copy(wv.at[pl.ds(0, 656)],
                    den_sp.at[pl.ds(roff + 7 * CHUNK, 656)])
    pltpu.sync_copy(featg.at[pl.ds(0, 656)],
                    num_sp.at[pl.ds(roff + 7 * CHUNK, 656)])
    plsc.subcore_barrier()

    ebase = _wid() * EPW

    def _chunk(k, _):
        base = ebase + k * CHUNK
        pltpu.sync_copy(elsrc_hbm.at[pl.ds(base, CHUNK)], elsv)
        pltpu.sync_copy(src_hbm.at[pl.ds(base, CHUNK)], srci.at[0])
        pltpu.sync_copy(dst_hbm.at[pl.ds(base, CHUNK)], dsti.at[0])
        pltpu.sync_copy(er_hbm.at[dsti.at[0]], erg)
        pltpu.sync_copy(emax_hbm.at[dsti.at[0]], emg)
        pltpu.sync_copy(feat_hbm.at[srci.at[0]], featg)

        def _w(j, _):
            sl = pl.ds(j * 16, 16)
            t0 = elsv[sl] + erg[sl]
            e = jnp.where(t0 >= 0.0, t0, 0.2 * t0)
            w = jnp.exp(e - emg[sl])
            wv[sl] = w
            for k in range(16):
                kk = j * 16 + k
                featg[kk] = featg[kk] * w[k]
            return 0

        lax.fori_loop(0, CHUNK // 16, _w, 0)

        pltpu.sync_copy(featg, num_sp.at[dsti.at[0]], add=True)
        pltpu.sync_copy(wv, den_sp.at[dsti.at[0]], add=True)
        return 0

    lax.fori_loop(0, NCHUNK, _chunk, 0)
    plsc.subcore_barrier()

    for t in range(7):
        pltpu.sync_copy(den_sp.at[pl.ds(roff + t * CHUNK, CHUNK)], wv)
        pltpu.sync_copy(wv, den_hbm.at[pl.ds(c * NPAD + roff + t * CHUNK,
                                             CHUNK)])
        pltpu.sync_copy(num_sp.at[pl.ds(roff + t * CHUNK, CHUNK)], featg)
        pltpu.sync_copy(featg, num_hbm.at[pl.ds(c * NPAD + roff + t * CHUNK,
                                                CHUNK)])
    pltpu.sync_copy(den_sp.at[pl.ds(roff + 7 * CHUNK, 656)],
                    wv.at[pl.ds(0, 656)])
    pltpu.sync_copy(wv.at[pl.ds(0, 656)],
                    den_hbm.at[pl.ds(c * NPAD + roff + 7 * CHUNK, 656)])
    pltpu.sync_copy(num_sp.at[pl.ds(roff + 7 * CHUNK, 656)],
                    featg.at[pl.ds(0, 656)])
    pltpu.sync_copy(featg.at[pl.ds(0, 656)],
                    num_hbm.at[pl.ds(c * NPAD + roff + 7 * CHUNK, 656)])


def _dense1_body(degp, w1, al, ar, feat_o, el_o, er_o):
    deg = degp[0] + degp[1]
    f = deg * w1[...]
    feat_o[...] = f
    el_o[...] = jnp.sum(f * al[...], axis=1, keepdims=True)
    er_o[...] = jnp.sum(f * ar[...], axis=1, keepdims=True)


def _dense1(degp, w1, al, ar):
    return pl.pallas_call(
        _dense1_body,
        grid=(GRID,),
        in_specs=[
            pl.BlockSpec((2, RBLK, 1), lambda i: (0, i, 0)),
            pl.BlockSpec((1, HID), lambda i: (0, 0)),
            pl.BlockSpec((1, HID), lambda i: (0, 0)),
            pl.BlockSpec((1, HID), lambda i: (0, 0)),
        ],
        out_specs=[
            pl.BlockSpec((RBLK, HID), lambda i: (i, 0)),
            pl.BlockSpec((RBLK, 1), lambda i: (i, 0)),
            pl.BlockSpec((RBLK, 1), lambda i: (i, 0)),
        ],
        out_shape=[
            jax.ShapeDtypeStruct((NPAD, HID), jnp.float32),
            jax.ShapeDtypeStruct((NPAD, 1), jnp.float32),
            jax.ShapeDtypeStruct((NPAD, 1), jnp.float32),
        ],
    )(degp, w1, al, ar)


def _dense23_body(nump, denp, b, w, al, ar, feat_o, el_o, er_o):
    num = nump[0] + nump[1]
    den = denp[0] + denp[1]
    safe = jnp.where(den > 0.0, den, 1.0)
    rst = jnp.where(den > 0.0, num / safe, 0.0) + b[...]
    h = jnp.maximum(rst, 0.0)
    f = jnp.dot(h, w[...], preferred_element_type=jnp.float32)
    feat_o[...] = f
    el_o[...] = jnp.sum(f * al[...], axis=1, keepdims=True)
    er_o[...] = jnp.sum(f * ar[...], axis=1, keepdims=True)


def _dense23(nump, denp, b, w, al, ar):
    return pl.pallas_call(
        _dense23_body,
        grid=(GRID,),
        in_specs=[
            pl.BlockSpec((2, RBLK, HID), lambda i: (0, i, 0)),
            pl.BlockSpec((2, RBLK, 1), lambda i: (0, i, 0)),
            pl.BlockSpec((1, HID), lambda i: (0, 0)),
            pl.BlockSpec((HID, HID), lambda i: (0, 0)),
            pl.BlockSpec((1, HID), lambda i: (0, 0)),
            pl.BlockSpec((1, HID), lambda i: (0, 0)),
        ],
        out_specs=[
            pl.BlockSpec((RBLK, HID), lambda i: (i, 0)),
            pl.BlockSpec((RBLK, 1), lambda i: (i, 0)),
            pl.BlockSpec((RBLK, 1), lambda i: (i, 0)),
        ],
        out_shape=[
            jax.ShapeDtypeStruct((NPAD, HID), jnp.float32),
            jax.ShapeDtypeStruct((NPAD, 1), jnp.float32),
            jax.ShapeDtypeStruct((NPAD, 1), jnp.float32),
        ],
    )(nump, denp, b, w, al, ar)


def _emax_body(pm, er, out):
    m = jnp.maximum(pm[0], pm[1]) + er[...]
    out[...] = jnp.where(m >= 0.0, m, 0.2 * m)


def _emax(pm, er):
    return pl.pallas_call(
        _emax_body,
        grid=(GRID,),
        in_specs=[
            pl.BlockSpec((2, RBLK, 1), lambda i: (0, i, 0)),
            pl.BlockSpec((RBLK, 1), lambda i: (i, 0)),
        ],
        out_specs=pl.BlockSpec((RBLK, 1), lambda i: (i, 0)),
        out_shape=jax.ShapeDtypeStruct((NPAD, 1), jnp.float32),
    )(pm, er)


def _final_body(nump, denp, b, wc, bc, out, acc):
    i = pl.program_id(0)
    num = nump[0] + nump[1]
    den = denp[0] + denp[1]
    safe = jnp.where(den > 0.0, den, 1.0)
    h = jnp.maximum(jnp.where(den > 0.0, num / safe, 0.0) + b[...], 0.0)
    rows = jax.lax.broadcasted_iota(jnp.int32, (RBLK, 1), 0) + i * RBLK
    h = jnp.where(rows < N_NODES, h, 0.0)

    @pl.when(i == 0)
    def _():
        acc[...] = jnp.zeros_like(acc)

    acc[...] += jnp.sum(h, axis=0, keepdims=True)

    @pl.when(i == GRID - 1)
    def _():
        hg = acc[...] * (1.0 / N_NODES)
        out[...] = (jnp.dot(hg, wc[...], preferred_element_type=jnp.float32)
                    + bc[...])


def _final(nump, denp, b, wc, bc):
    return pl.pallas_call(
        _final_body,
        grid=(GRID,),
        in_specs=[
            pl.BlockSpec((2, RBLK, HID), lambda i: (0, i, 0)),
            pl.BlockSpec((2, RBLK, 1), lambda i: (0, i, 0)),
            pl.BlockSpec((1, HID), lambda i: (0, 0)),
            pl.BlockSpec((HID, N_CLASSES), lambda i: (0, 0)),
            pl.BlockSpec((1, N_CLASSES), lambda i: (0, 0)),
        ],
        out_specs=pl.BlockSpec((1, N_CLASSES), lambda i: (0, 0)),
        out_shape=jax.ShapeDtypeStruct((1, N_CLASSES), jnp.float32),
        scratch_shapes=[pltpu.VMEM((1, HID), jnp.float32)],
    )(nump, denp, b, wc, bc)


def _gat_edge_phase(src, dst, feat, el, er):
    pm, elsrc = _maxel_kernel(src, dst, el.reshape(NPAD))
    emax = _emax(pm.reshape(2, NPAD, 1), er)
    nump, denp = _edge_kernel(src, dst, elsrc, er.reshape(NPAD),
                              emax.reshape(NPAD), feat)
    return nump.reshape(2, NPAD, HID), denp.reshape(2, NPAD, 1)


def kernel(edge_index, W1, al1, ar1, b1, W2, al2, ar2, b2,
           W3, al3, ar3, b3, Wc, bc):
    src = edge_index[0]
    dst = edge_index[1]

    degp = _deg_kernel(dst).reshape(2, NPAD, 1)
    feat1, el1, er1 = _dense1(degp, W1, al1.reshape(1, HID),
                              ar1.reshape(1, HID))
    nump, denp = _gat_edge_phase(src, dst, feat1, el1, er1)

    feat2, el2, er2 = _dense23(nump, denp, b1.reshape(1, HID), W2,
                               al2.reshape(1, HID), ar2.reshape(1, HID))
    nump, denp = _gat_edge_phase(src, dst, feat2, el2, er2)

    feat3, el3, er3 = _dense23(nump, denp, b2.reshape(1, HID), W3,
                               al3.reshape(1, HID), ar3.reshape(1, HID))
    nump, denp = _gat_edge_phase(src, dst, feat3, el3, er3)

    return _final(nump, denp, b3.reshape(1, HID), Wc,
                  bc.reshape(1, N_CLASSES))


# speedup vs baseline: 74.9094x; 1.2505x over previous
import functools

import jax
import jax.numpy as jnp
from jax import lax
from jax.experimental import pallas as pl
from jax.experimental.pallas import tpu as pltpu
from jax.experimental.pallas import tpu_sc as plsc

N_NODES = 100000
N_EDGES = 3200000
HID = 16
N_CLASSES = 10

NCORE = 2
NSUB = 16
NW = NCORE * NSUB
EPW = N_EDGES // NW
CHUNK = 400
NCHUNK = EPW // CHUNK
NPAIR = NCHUNK // 2

RPT = 6256
NPAD = RPT * NSUB

RBLK = 2176
GRID = NPAD // RBLK

NEG = -3.0e38


def _wid():
    return lax.axis_index("s") * NCORE + lax.axis_index("c")


_MESH = plsc.VectorSubcoreMesh(core_axis_name="c", subcore_axis_name="s")
_CP = pltpu.CompilerParams(needs_layout_passes=False,
                           use_tc_tiling_on_sc=False)


@functools.partial(
    pl.kernel,
    mesh=_MESH,
    compiler_params=_CP,
    out_type=jax.ShapeDtypeStruct((NCORE * NPAD,), jnp.float32),
    scratch_types=[
        pltpu.VMEM((1, CHUNK), jnp.int32),
        pltpu.VMEM((1, CHUNK), jnp.int32),
        pltpu.VMEM((CHUNK,), jnp.float32),
        pltpu.VMEM((RPT,), jnp.float32),
        pltpu.VMEM_SHARED((NPAD,), jnp.float32),
        pltpu.SemaphoreType.DMA,
        pltpu.SemaphoreType.DMA,
        pltpu.SemaphoreType.DMA,
        pltpu.SemaphoreType.DMA,
    ],
)
def _deg_kernel(dst_hbm, out_hbm, dsti0, dsti1, ones_v, zb, deg_sp,
                sl0, sl1, ss0, ss1):
    c = lax.axis_index("c")
    s = lax.axis_index("s")
    roff = s * RPT

    def _fill(i, _):
        zb[pl.ds(i * 16, 16)] = jnp.zeros((16,), jnp.float32)
        return 0

    lax.fori_loop(0, RPT // 16, _fill, 0)

    def _fill1(i, _):
        ones_v[pl.ds(i * 16, 16)] = jnp.ones((16,), jnp.float32)
        return 0

    lax.fori_loop(0, CHUNK // 16, _fill1, 0)
    pltpu.sync_copy(zb, deg_sp.at[pl.ds(roff, RPT)])
    plsc.subcore_barrier()

    ebase = _wid() * EPW

    def _pair(kk, _):
        b0 = ebase + (2 * kk) * CHUNK
        b1 = b0 + CHUNK
        hl0 = pltpu.async_copy(dst_hbm.at[pl.ds(b0, CHUNK)], dsti0.at[0], sl0)
        hl1 = pltpu.async_copy(dst_hbm.at[pl.ds(b1, CHUNK)], dsti1.at[0], sl1)
        hl0.wait()
        hs0 = pltpu.async_copy(ones_v, deg_sp.at[dsti0.at[0]], ss0, add=True)
        hl1.wait()
        hs1 = pltpu.async_copy(ones_v, deg_sp.at[dsti1.at[0]], ss1, add=True)
        hs0.wait()
        hs1.wait()
        return 0

    lax.fori_loop(0, NPAIR, _pair, 0)
    plsc.subcore_barrier()
    pltpu.sync_copy(deg_sp.at[pl.ds(roff, RPT)], zb)
    pltpu.sync_copy(zb, out_hbm.at[pl.ds(c * NPAD + roff, RPT)])


def _segmax_update(mx, dstv, elv):
    def _vec(j, _):
        idx = dstv[pl.ds(j * 16, 16)]
        ev = elv[pl.ds(j * 16, 16)]
        cur = plsc.load_gather(mx, [idx])

        def _cond(carry):
            return jnp.any(ev > carry)

        def _body(carry):
            plsc.store_scatter(mx, [idx], ev, mask=ev > carry)
            return plsc.load_gather(mx, [idx])

        lax.while_loop(_cond, _body, cur)
        return 0

    lax.fori_loop(0, CHUNK // 16, _vec, 0)


@functools.partial(
    pl.kernel,
    mesh=_MESH,
    compiler_params=_CP,
    out_type=jax.ShapeDtypeStruct((NCORE * NPAD,), jnp.float32),
    scratch_types=[
        pltpu.VMEM((1, CHUNK), jnp.int32),
        pltpu.VMEM((1, CHUNK), jnp.int32),
        pltpu.VMEM((CHUNK,), jnp.int32),
        pltpu.VMEM((CHUNK,), jnp.int32),
        pltpu.VMEM((CHUNK,), jnp.float32),
        pltpu.VMEM((CHUNK,), jnp.float32),
        pltpu.VMEM((NPAD,), jnp.float32),
        pltpu.VMEM((RPT,), jnp.float32),
        pltpu.VMEM((RPT,), jnp.float32),
        pltpu.HBM((NW * NPAD,), jnp.float32),
        pltpu.SemaphoreType.DMA,
        pltpu.SemaphoreType.DMA,
        pltpu.SemaphoreType.DMA,
        pltpu.SemaphoreType.DMA,
    ],
)
def _maxel_kernel(src_hbm, dst_hbm, el_hbm, pm_hbm,
                  srci0, srci1, dstv0, dstv1, elv0, elv1,
                  mx, acc, tmp, stage, sl0, sl1, sg0, sg1):
    c = lax.axis_index("c")
    s = lax.axis_index("s")

    def _fill(i, _):
        mx[pl.ds(i * 16, 16)] = jnp.full((16,), NEG, jnp.float32)
        return 0

    lax.fori_loop(0, NPAD // 16, _fill, 0)

    ebase = _wid() * EPW

    def _pair(kk, _):
        b0 = ebase + (2 * kk) * CHUNK
        b1 = b0 + CHUNK
        hl0a = pltpu.async_copy(dst_hbm.at[pl.ds(b0, CHUNK)], dstv0, sl0)
        hl0b = pltpu.async_copy(src_hbm.at[pl.ds(b0, CHUNK)], srci0.at[0], sl0)
        hl1a = pltpu.async_copy(dst_hbm.at[pl.ds(b1, CHUNK)], dstv1, sl1)
        hl1b = pltpu.async_copy(src_hbm.at[pl.ds(b1, CHUNK)], srci1.at[0], sl1)
        hl0a.wait()
        hl0b.wait()
        hg0 = pltpu.async_copy(el_hbm.at[srci0.at[0]], elv0, sg0)
        hl1a.wait()
        hl1b.wait()
        hg1 = pltpu.async_copy(el_hbm.at[srci1.at[0]], elv1, sg1)
        hg0.wait()
        _segmax_update(mx, dstv0, elv0)
        hg1.wait()
        _segmax_update(mx, dstv1, elv1)
        return 0

    lax.fori_loop(0, NPAIR, _pair, 0)

    pltpu.sync_copy(mx, stage.at[pl.ds((c * NSUB + s) * NPAD, NPAD)])
    plsc.subcore_barrier()

    roff = s * RPT
    for t in range(NSUB):
        if t == 0:
            pltpu.sync_copy(stage.at[pl.ds(c * NSUB * NPAD + roff, RPT)], acc)
        else:
            pltpu.sync_copy(stage.at[pl.ds((c * NSUB + t) * NPAD + roff, RPT)],
                            tmp)

            def _red(j, _):
                acc[pl.ds(j * 16, 16)] = jnp.maximum(
                    acc[pl.ds(j * 16, 16)], tmp[pl.ds(j * 16, 16)])
                return 0

            lax.fori_loop(0, RPT // 16, _red, 0)
    pltpu.sync_copy(acc, pm_hbm.at[pl.ds(c * NPAD + roff, RPT)])


@functools.partial(
    pl.kernel,
    mesh=_MESH,
    compiler_params=_CP,
    out_type=(
        jax.ShapeDtypeStruct((NCORE * NPAD, HID), jnp.float32),
        jax.ShapeDtypeStruct((NCORE * NPAD,), jnp.float32),
    ),
    scratch_types=[
        pltpu.VMEM((1, CHUNK), jnp.int32),
        pltpu.VMEM((1, CHUNK), jnp.int32),
        pltpu.VMEM((1, CHUNK), jnp.int32),
        pltpu.VMEM((1, CHUNK), jnp.int32),
        pltpu.VMEM((CHUNK,), jnp.float32),
        pltpu.VMEM((CHUNK,), jnp.float32),
        pltpu.VMEM((CHUNK,), jnp.float32),
        pltpu.VMEM((CHUNK,), jnp.float32),
        pltpu.VMEM((CHUNK,), jnp.float32),
        pltpu.VMEM((CHUNK,), jnp.float32),
        pltpu.VMEM((CHUNK,), jnp.float32),
        pltpu.VMEM((CHUNK,), jnp.float32),
        pltpu.VMEM((CHUNK, HID), jnp.float32),
        pltpu.VMEM((CHUNK, HID), jnp.float32),
        pltpu.VMEM_SHARED((NPAD, HID), jnp.float32),
        pltpu.VMEM_SHARED((NPAD,), jnp.float32),
        pltpu.SemaphoreType.DMA,
        pltpu.SemaphoreType.DMA,
        pltpu.SemaphoreType.DMA,
        pltpu.SemaphoreType.DMA,
        pltpu.SemaphoreType.DMA,
        pltpu.SemaphoreType.DMA,
    ],
)
def _edge_kernel(src_hbm, dst_hbm, el_hbm, er_hbm, emax_hbm, feat_hbm,
                 num_hbm, den_hbm,
                 srci0, srci1, dsti0, dsti1,
                 elg0, elg1, erg0, erg1, emg0, emg1, wv0, wv1,
                 featg0, featg1, num_sp, den_sp,
                 sl0, sl1, sg0, sg1, ss0, ss1):
    c = lax.axis_index("c")
    s = lax.axis_index("s")
    roff = s * RPT

    def _fillzr(i, _):
        featg0[i] = jnp.zeros((HID,), jnp.float32)
        return 0

    lax.fori_loop(0, CHUNK, _fillzr, 0)

    def _fillz(i, _):
        wv0[pl.ds(i * 16, 16)] = jnp.zeros((16,), jnp.float32)
        return 0

    lax.fori_loop(0, CHUNK // 16, _fillz, 0)

    for t in range(15):
        pltpu.sync_copy(wv0, den_sp.at[pl.ds(roff + t * CHUNK, CHUNK)])
        pltpu.sync_copy(featg0, num_sp.at[pl.ds(roff + t * CHUNK, CHUNK)])
    pltpu.sync_copy(wv0.at[pl.ds(0, 256)],
                    den_sp.at[pl.ds(roff + 15 * CHUNK, 256)])
    pltpu.sync_copy(featg0.at[pl.ds(0, 256)],
                    num_sp.at[pl.ds(roff + 15 * CHUNK, 256)])
    plsc.subcore_barrier()

    ebase = _wid() * EPW

    def _compute(elg, erg, emg, wv, featg):
        def _w(j, _):
            sl = pl.ds(j * 16, 16)
            t0 = elg[sl] + erg[sl]
            e = jnp.where(t0 >= 0.0, t0, 0.2 * t0)
            w = jnp.exp(e - emg[sl])
            wv[sl] = w
            for k in range(16):
                kk = j * 16 + k
                featg[kk] = featg[kk] * w[k]
            return 0

        lax.fori_loop(0, CHUNK // 16, _w, 0)

    def _fire_loads(b, srci, dsti, sl):
        ha = pltpu.async_copy(src_hbm.at[pl.ds(b, CHUNK)], srci.at[0], sl)
        hb = pltpu.async_copy(dst_hbm.at[pl.ds(b, CHUNK)], dsti.at[0], sl)
        return ha, hb

    def _fire_gathers(srci, dsti, elg, erg, emg, featg, sg):
        h1 = pltpu.async_copy(el_hbm.at[srci.at[0]], elg, sg)
        h2 = pltpu.async_copy(er_hbm.at[dsti.at[0]], erg, sg)
        h3 = pltpu.async_copy(emax_hbm.at[dsti.at[0]], emg, sg)
        h4 = pltpu.async_copy(feat_hbm.at[srci.at[0]], featg, sg)
        return h1, h2, h3, h4

    def _pair(kk, _):
        b0 = ebase + (2 * kk) * CHUNK
        b1 = b0 + CHUNK
        l0 = _fire_loads(b0, srci0, dsti0, sl0)
        l1 = _fire_loads(b1, srci1, dsti1, sl1)
        for h in l0:
            h.wait()
        g0 = _fire_gathers(srci0, dsti0, elg0, erg0, emg0, featg0, sg0)
        for h in l1:
            h.wait()
        g1 = _fire_gathers(srci1, dsti1, elg1, erg1, emg1, featg1, sg1)
        for h in g0:
            h.wait()
        _compute(elg0, erg0, emg0, wv0, featg0)
        hs0a = pltpu.async_copy(featg0, num_sp.at[dsti0.at[0]], ss0, add=True)
        hs0b = pltpu.async_copy(wv0, den_sp.at[dsti0.at[0]], ss0, add=True)
        for h in g1:
            h.wait()
        _compute(elg1, erg1, emg1, wv1, featg1)
        hs1a = pltpu.async_copy(featg1, num_sp.at[dsti1.at[0]], ss1, add=True)
        hs1b = pltpu.async_copy(wv1, den_sp.at[dsti1.at[0]], ss1, add=True)
        hs0a.wait()
        hs0b.wait()
        hs1a.wait()
        hs1b.wait()
        return 0

    lax.fori_loop(0, NPAIR, _pair, 0)
    plsc.subcore_barrier()

    for t in range(15):
        pltpu.sync_copy(den_sp.at[pl.ds(roff + t * CHUNK, CHUNK)], wv0)
        pltpu.sync_copy(wv0, den_hbm.at[pl.ds(c * NPAD + roff + t * CHUNK,
                                              CHUNK)])
        pltpu.sync_copy(num_sp.at[pl.ds(roff + t * CHUNK, CHUNK)], featg0)
        pltpu.sync_copy(featg0, num_hbm.at[pl.ds(c * NPAD + roff + t * CHUNK,
                                                 CHUNK)])
    pltpu.sync_copy(den_sp.at[pl.ds(roff + 15 * CHUNK, 256)],
                    wv0.at[pl.ds(0, 256)])
    pltpu.sync_copy(wv0.at[pl.ds(0, 256)],
                    den_hbm.at[pl.ds(c * NPAD + roff + 15 * CHUNK, 256)])
    pltpu.sync_copy(num_sp.at[pl.ds(roff + 15 * CHUNK, 256)],
                    featg0.at[pl.ds(0, 256)])
    pltpu.sync_copy(featg0.at[pl.ds(0, 256)],
                    num_hbm.at[pl.ds(c * NPAD + roff + 15 * CHUNK, 256)])


def _dense1_body(degp, w1, al, ar, feat_o, el_o, er_o):
    deg = degp[0] + degp[1]
    f = deg * w1[...]
    feat_o[...] = f
    el_o[...] = jnp.sum(f * al[...], axis=1, keepdims=True)
    er_o[...] = jnp.sum(f * ar[...], axis=1, keepdims=True)


def _dense1(degp, w1, al, ar):
    return pl.pallas_call(
        _dense1_body,
        grid=(GRID,),
        in_specs=[
            pl.BlockSpec((2, RBLK, 1), lambda i: (0, i, 0)),
            pl.BlockSpec((1, HID), lambda i: (0, 0)),
            pl.BlockSpec((1, HID), lambda i: (0, 0)),
            pl.BlockSpec((1, HID), lambda i: (0, 0)),
        ],
        out_specs=[
            pl.BlockSpec((RBLK, HID), lambda i: (i, 0)),
            pl.BlockSpec((RBLK, 1), lambda i: (i, 0)),
            pl.BlockSpec((RBLK, 1), lambda i: (i, 0)),
        ],
        out_shape=[
            jax.ShapeDtypeStruct((NPAD, HID), jnp.float32),
            jax.ShapeDtypeStruct((NPAD, 1), jnp.float32),
            jax.ShapeDtypeStruct((NPAD, 1), jnp.float32),
        ],
    )(degp, w1, al, ar)


def _dense23_body(nump, denp, b, w, al, ar, feat_o, el_o, er_o):
    num = nump[0] + nump[1]
    den = denp[0] + denp[1]
    safe = jnp.where(den > 0.0, den, 1.0)
    rst = jnp.where(den > 0.0, num / safe, 0.0) + b[...]
    h = jnp.maximum(rst, 0.0)
    f = jnp.dot(h, w[...], preferred_element_type=jnp.float32)
    feat_o[...] = f
    el_o[...] = jnp.sum(f * al[...], axis=1, keepdims=True)
    er_o[...] = jnp.sum(f * ar[...], axis=1, keepdims=True)


def _dense23(nump, denp, b, w, al, ar):
    return pl.pallas_call(
        _dense23_body,
        grid=(GRID,),
        in_specs=[
            pl.BlockSpec((2, RBLK, HID), lambda i: (0, i, 0)),
            pl.BlockSpec((2, RBLK, 1), lambda i: (0, i, 0)),
            pl.BlockSpec((1, HID), lambda i: (0, 0)),
            pl.BlockSpec((HID, HID), lambda i: (0, 0)),
            pl.BlockSpec((1, HID), lambda i: (0, 0)),
            pl.BlockSpec((1, HID), lambda i: (0, 0)),
        ],
        out_specs=[
            pl.BlockSpec((RBLK, HID), lambda i: (i, 0)),
            pl.BlockSpec((RBLK, 1), lambda i: (i, 0)),
            pl.BlockSpec((RBLK, 1), lambda i: (i, 0)),
        ],
        out_shape=[
            jax.ShapeDtypeStruct((NPAD, HID), jnp.float32),
            jax.ShapeDtypeStruct((NPAD, 1), jnp.float32),
            jax.ShapeDtypeStruct((NPAD, 1), jnp.float32),
        ],
    )(nump, denp, b, w, al, ar)


def _emax_body(pm, er, out):
    m = jnp.maximum(pm[0], pm[1]) + er[...]
    out[...] = jnp.where(m >= 0.0, m, 0.2 * m)


def _emax(pm, er):
    return pl.pallas_call(
        _emax_body,
        grid=(GRID,),
        in_specs=[
            pl.BlockSpec((2, RBLK, 1), lambda i: (0, i, 0)),
            pl.BlockSpec((RBLK, 1), lambda i: (i, 0)),
        ],
        out_specs=pl.BlockSpec((RBLK, 1), lambda i: (i, 0)),
        out_shape=jax.ShapeDtypeStruct((NPAD, 1), jnp.float32),
    )(pm, er)


def _final_body(nump, denp, b, wc, bc, out, acc):
    i = pl.program_id(0)
    num = nump[0] + nump[1]
    den = denp[0] + denp[1]
    safe = jnp.where(den > 0.0, den, 1.0)
    h = jnp.maximum(jnp.where(den > 0.0, num / safe, 0.0) + b[...], 0.0)
    rows = jax.lax.broadcasted_iota(jnp.int32, (RBLK, 1), 0) + i * RBLK
    h = jnp.where(rows < N_NODES, h, 0.0)

    @pl.when(i == 0)
    def _():
        acc[...] = jnp.zeros_like(acc)

    acc[...] += jnp.sum(h, axis=0, keepdims=True)

    @pl.when(i == GRID - 1)
    def _():
        hg = acc[...] * (1.0 / N_NODES)
        out[...] = (jnp.dot(hg, wc[...], preferred_element_type=jnp.float32)
                    + bc[...])


def _final(nump, denp, b, wc, bc):
    return pl.pallas_call(
        _final_body,
        grid=(GRID,),
        in_specs=[
            pl.BlockSpec((2, RBLK, HID), lambda i: (0, i, 0)),
            pl.BlockSpec((2, RBLK, 1), lambda i: (0, i, 0)),
            pl.BlockSpec((1, HID), lambda i: (0, 0)),
            pl.BlockSpec((HID, N_CLASSES), lambda i: (0, 0)),
            pl.BlockSpec((1, N_CLASSES), lambda i: (0, 0)),
        ],
        out_specs=pl.BlockSpec((1, N_CLASSES), lambda i: (0, 0)),
        out_shape=jax.ShapeDtypeStruct((1, N_CLASSES), jnp.float32),
        scratch_shapes=[pltpu.VMEM((1, HID), jnp.float32)],
    )(nump, denp, b, wc, bc)


def _gat_edge_phase(src, dst, feat, el, er):
    pm = _maxel_kernel(src, dst, el.reshape(NPAD))
    emax = _emax(pm.reshape(2, NPAD, 1), er)
    nump, denp = _edge_kernel(src, dst, el.reshape(NPAD), er.reshape(NPAD),
                              emax.reshape(NPAD), feat)
    return nump.reshape(2, NPAD, HID), denp.reshape(2, NPAD, 1)


def kernel(edge_index, W1, al1, ar1, b1, W2, al2, ar2, b2,
           W3, al3, ar3, b3, Wc, bc):
    src = edge_index[0]
    dst = edge_index[1]

    degp = _deg_kernel(dst).reshape(2, NPAD, 1)
    feat1, el1, er1 = _dense1(degp, W1, al1.reshape(1, HID),
                              ar1.reshape(1, HID))
    nump, denp = _gat_edge_phase(src, dst, feat1, el1, er1)

    feat2, el2, er2 = _dense23(nump, denp, b1.reshape(1, HID), W2,
                               al2.reshape(1, HID), ar2.reshape(1, HID))
    nump, denp = _gat_edge_phase(src, dst, feat2, el2, er2)

    feat3, el3, er3 = _dense23(nump, denp, b2.reshape(1, HID), W3,
                               al3.reshape(1, HID), ar3.reshape(1, HID))
    nump, denp = _gat_edge_phase(src, dst, feat3, el3, er3)

    return _final(nump, denp, b3.reshape(1, HID), Wc,
                  bc.reshape(1, N_CLASSES))
